# Initial kernel scaffold; baseline (speedup 1.0000x reference)
import jax
import jax.numpy as jnp
from jax.experimental import pallas as pl


def kernel(user_x, item_x, user_edge_index, item_edge_index, u_Wl1, u_bl1, u_Wr1, u_Wl2, u_bl2, u_Wr2, i_Wl1, i_bl1, i_Wr1, i_Wl2, i_bl2, i_Wr2, s_W, s_b):
    raise NotImplementedError("write your pallas kernel here")


# speedup vs baseline: 7.4003x; 7.4003x over previous
import functools

import jax
import jax.numpy as jnp
from jax import lax
from jax.experimental import pallas as pl
from jax.experimental.pallas import tpu as pltpu
from jax.experimental.pallas import tpu_sc as plsc

N = 10000
E = 320000
IN_DIM = 128
HID = 256
EMB = 64

NC = 2
NS = 16
C = 125
EPS = E // NS
CH = EPS // C
NP = 10240
RPS = NP // NS
CZ = 128
NWB = RPS // CZ

_f32 = jnp.float32


D = EMB


def _make_sc_segsum(num_passes, with_deg):
  mesh = plsc.VectorSubcoreMesh(core_axis_name="c", subcore_axis_name="s")

  out_type = [jax.ShapeDtypeStruct((NP, D), _f32)] * (2 * num_passes)
  if with_deg:
    out_type += [jax.ShapeDtypeStruct((NP, 16), _f32),
                 jax.ShapeDtypeStruct((NP, 16), _f32)]

  scratch = [
      pltpu.VMEM((CH, C), jnp.int32),
      pltpu.VMEM((CH, C), jnp.int32),
      pltpu.VMEM((C, D), _f32),
      pltpu.VMEM((CZ, D), _f32),
      pltpu.SemaphoreType.DMA,
      pltpu.VMEM_SHARED((NP, D), _f32),
  ]
  if with_deg:
    scratch += [
        pltpu.VMEM((C, 16), _f32),
        pltpu.VMEM((CZ, 16), _f32),
        pltpu.VMEM_SHARED((NP, 16), _f32),
    ]

  @functools.partial(pl.kernel, out_type=out_type, mesh=mesh,
                     scratch_types=scratch,
                     compiler_params=pltpu.CompilerParams(
                         use_tc_tiling_on_sc=False))
  def sc_kernel(*refs):
    pos = 0
    xs_u = refs[pos:pos + num_passes]; pos += num_passes
    xs_i = refs[pos:pos + num_passes]; pos += num_passes
    su, du, si, di, zfeat_hbm = refs[pos:pos + 5]; pos += 5
    if with_deg:
      z16_hbm, ones_hbm = refs[pos:pos + 2]; pos += 2
    aggs_u = refs[pos:pos + num_passes]; pos += num_passes
    aggs_i = refs[pos:pos + num_passes]; pos += num_passes
    if with_deg:
      deg_u, deg_i = refs[pos:pos + 2]; pos += 2
    srcv, dstv, rowb, zfv, gsem, acc = refs[pos:pos + 6]; pos += 6
    if with_deg:
      onesv, z16v, dacc = refs[pos:pos + 3]; pos += 3

    c = lax.axis_index("c")
    s = lax.axis_index("s")
    base = s * RPS

    pltpu.sync_copy(zfeat_hbm, zfv)
    if with_deg:
      pltpu.sync_copy(z16_hbm, z16v)
      pltpu.sync_copy(ones_hbm, onesv)

    @pl.when(c == 0)
    def _():
      pltpu.sync_copy(su.at[s], srcv)
      pltpu.sync_copy(du.at[s], dstv)

    @pl.when(c == 1)
    def _():
      pltpu.sync_copy(si.at[s], srcv)
      pltpu.sync_copy(di.at[s], dstv)

    for p in range(num_passes):
      first = (p == 0)
      for k in range(NWB):
        pltpu.sync_copy(zfv, acc.at[pl.ds(base + k * CZ, CZ)])
        if with_deg and first:
          pltpu.sync_copy(z16v, dacc.at[pl.ds(base + k * CZ, CZ)])
      plsc.subcore_barrier()

      def run(x_hbm, do_deg):
        def step(j, carry):
          pltpu.async_copy(x_hbm.at[srcv.at[j]], rowb, gsem).wait()
          pltpu.sync_copy(rowb, acc.at[dstv.at[j]], add=True)
          if do_deg:
            pltpu.sync_copy(onesv, dacc.at[dstv.at[j]], add=True)
          return carry

        lax.fori_loop(0, CH, step, 0)

      @pl.when(c == 0)
      def _():
        run(xs_u[p], with_deg and first)

      @pl.when(c == 1)
      def _():
        run(xs_i[p], with_deg and first)

      plsc.subcore_barrier()

      def writeback(agg_hbm, deg_hbm):
        for k in range(NWB):
          off = base + k * CZ
          pltpu.sync_copy(acc.at[pl.ds(off, CZ)], zfv)
          pltpu.sync_copy(zfv, agg_hbm.at[pl.ds(off, CZ)])
          if deg_hbm is not None:
            pltpu.sync_copy(dacc.at[pl.ds(off, CZ)], z16v)
            pltpu.sync_copy(z16v, deg_hbm.at[pl.ds(off, CZ)])

      @pl.when(c == 0)
      def _():
        writeback(aggs_u[p], deg_u if (with_deg and first) else None)

      @pl.when(c == 1)
      def _():
        writeback(aggs_i[p], deg_i if (with_deg and first) else None)

      if p + 1 < num_passes:
        pltpu.sync_copy(zfeat_hbm, zfv)

  return sc_kernel


_sc_segsum_l1 = _make_sc_segsum(2, with_deg=True)
_sc_segsum_l2 = _make_sc_segsum(1, with_deg=False)


_BT = 1000
_GT = N // _BT


def _dot_t(a, w):
  return lax.dot_general(a, w, (((1,), (1,)), ((), ())),
                         preferred_element_type=_f32)


def _tc_mid_body(agg_u0, agg_u1, deg_u, xu, agg_i0, agg_i1, deg_i, xi,
                 uWl1, ubl1, uWr1, uWl2, ubl2, uWr2,
                 iWl1, ibl1, iWr1, iWl2, ibl2, iWr2,
                 p_u, r_u, p_i, r_i):
  def enc(agg0, agg1, deg, x, Wl1, bl1, Wr1, Wl2, bl2, Wr2, p_out, r_out):
    d = jnp.maximum(deg[...][:, :1], 1.0)
    mean = jnp.concatenate([agg0[...], agg1[...]], axis=1) / d
    h = jnp.maximum(_dot_t(mean, Wl1[...]) + bl1[...] + _dot_t(x[...], Wr1[...]),
                    0.0)
    p_out[...] = _dot_t(h, Wl2[...])
    r_out[...] = _dot_t(h, Wr2[...]) + bl2[...]

  enc(agg_u0, agg_u1, deg_u, xu, uWl1, ubl1, uWr1, uWl2, ubl2, uWr2, p_u, r_u)
  enc(agg_i0, agg_i1, deg_i, xi, iWl1, ibl1, iWr1, iWl2, ibl2, iWr2, p_i, r_i)


def _row_spec(d):
  return pl.BlockSpec((_BT, d), lambda i: (i, 0))


def _full_spec(shape):
  nd = len(shape)
  return pl.BlockSpec(shape, lambda i: (0,) * nd)


def _tc_mid(agg_u0, agg_u1, deg_u, xu, agg_i0, agg_i1, deg_i, xi, wu, wi):
  w_specs = [_full_spec(w.shape) for w in (wu + wi)]
  return pl.pallas_call(
      _tc_mid_body,
      grid=(_GT,),
      in_specs=[_row_spec(EMB), _row_spec(EMB), _row_spec(16), _row_spec(IN_DIM),
                _row_spec(EMB), _row_spec(EMB), _row_spec(16), _row_spec(IN_DIM)]
               + w_specs,
      out_specs=[_row_spec(EMB)] * 4,
      out_shape=[jax.ShapeDtypeStruct((N, EMB), _f32)] * 4,
  )(agg_u0, agg_u1, deg_u, xu, agg_i0, agg_i1, deg_i, xi, *wu, *wi)


def _tc_head_body(a2u, deg_u, ru, a2i, deg_i, ri, sW, sb, out):
  eu = a2u[...] / jnp.maximum(deg_u[...][:, :1], 1.0) + ru[...]
  ei = a2i[...] / jnp.maximum(deg_i[...][:, :1], 1.0) + ri[...]
  w = sW[...]
  z = _dot_t(eu, w[:, :EMB]) + _dot_t(ei, w[:, EMB:]) + sb[...]
  out[...] = 1.0 / (1.0 + jnp.exp(-z))


def _tc_head(a2u, deg_u, ru, a2i, deg_i, ri, sW, sb):
  return pl.pallas_call(
      _tc_head_body,
      grid=(_GT,),
      in_specs=[_row_spec(EMB), _row_spec(16), _row_spec(EMB),
                _row_spec(EMB), _row_spec(16), _row_spec(EMB),
                _full_spec((1, 2 * EMB)), _full_spec((1, 1))],
      out_specs=_row_spec(1),
      out_shape=jax.ShapeDtypeStruct((N, 1), _f32),
  )(a2u, deg_u, ru, a2i, deg_i, ri, sW, sb)


def kernel(user_x, item_x, user_edge_index, item_edge_index,
           u_Wl1, u_bl1, u_Wr1, u_Wl2, u_bl2, u_Wr2,
           i_Wl1, i_bl1, i_Wr1, i_Wl2, i_bl2, i_Wr2,
           s_W, s_b):
  def edges(ei):
    src = ei[0].astype(jnp.int32).reshape(NS, CH, C)
    dst = ei[1].astype(jnp.int32).reshape(NS, CH, C)
    return src, dst

  su, du = edges(user_edge_index)
  si, di = edges(item_edge_index)

  z64 = jnp.zeros((CZ, D), _f32)
  z16 = jnp.zeros((CZ, 16), _f32)
  ones16 = jnp.ones((C, 16), _f32)

  xu0, xu1 = user_x[:, :D], user_x[:, D:]
  xi0, xi1 = item_x[:, :D], item_x[:, D:]

  agg_u0, agg_u1, agg_i0, agg_i1, deg_u, deg_i = _sc_segsum_l1(
      xu0, xu1, xi0, xi1, su, du, si, di, z64, z16, ones16)

  wu = (u_Wl1, u_bl1.reshape(1, HID), u_Wr1,
        u_Wl2, u_bl2.reshape(1, EMB), u_Wr2)
  wi = (i_Wl1, i_bl1.reshape(1, HID), i_Wr1,
        i_Wl2, i_bl2.reshape(1, EMB), i_Wr2)
  p_u, r_u, p_i, r_i = _tc_mid(agg_u0, agg_u1, deg_u, user_x,
                               agg_i0, agg_i1, deg_i, item_x, wu, wi)

  agg2_u, agg2_i = _sc_segsum_l2(p_u, p_i, su, du, si, di, z64)

  return _tc_head(agg2_u, deg_u, r_u, agg2_i, deg_i, r_i,
                  s_W, s_b.reshape(1, 1))


# speedup vs baseline: 11.5223x; 1.5570x over previous
import functools

import jax
import jax.numpy as jnp
from jax import lax
from jax.experimental import pallas as pl
from jax.experimental.pallas import tpu as pltpu
from jax.experimental.pallas import tpu_sc as plsc

N = 10000
E = 320000
IN_DIM = 128
HID = 256
EMB = 64

NC = 2
NS = 16
C = 125
EPS = E // NS
CH = EPS // C
NP = 10240
RPS = NP // NS
CZ = 128
NWB = RPS // CZ

_f32 = jnp.float32


D = EMB


def _make_sc_segsum(num_passes, with_deg):
  mesh = plsc.VectorSubcoreMesh(core_axis_name="c", subcore_axis_name="s")

  out_type = [jax.ShapeDtypeStruct((NP, D), _f32)] * (2 * num_passes)
  if with_deg:
    out_type += [jax.ShapeDtypeStruct((NP, 16), _f32),
                 jax.ShapeDtypeStruct((NP, 16), _f32)]

  scratch = [
      pltpu.VMEM((CH, C), jnp.int32),
      pltpu.VMEM((CH, C), jnp.int32),
      pltpu.VMEM((C, D), _f32),
      pltpu.VMEM((C, D), _f32),
      pltpu.VMEM((CZ, D), _f32),
      pltpu.SemaphoreType.DMA,
      pltpu.SemaphoreType.DMA,
      pltpu.VMEM_SHARED((NP, D), _f32),
  ]
  if with_deg:
    scratch += [
        pltpu.VMEM((C, 16), _f32),
        pltpu.VMEM((CZ, 16), _f32),
        pltpu.VMEM_SHARED((NP, 16), _f32),
    ]

  @functools.partial(pl.kernel, out_type=out_type, mesh=mesh,
                     scratch_types=scratch,
                     compiler_params=pltpu.CompilerParams(
                         use_tc_tiling_on_sc=False))
  def sc_kernel(*refs):
    pos = 0
    xs_u = refs[pos:pos + num_passes]; pos += num_passes
    xs_i = refs[pos:pos + num_passes]; pos += num_passes
    su, du, si, di, zfeat_hbm = refs[pos:pos + 5]; pos += 5
    if with_deg:
      z16_hbm, ones_hbm = refs[pos:pos + 2]; pos += 2
    aggs_u = refs[pos:pos + num_passes]; pos += num_passes
    aggs_i = refs[pos:pos + num_passes]; pos += num_passes
    if with_deg:
      deg_u, deg_i = refs[pos:pos + 2]; pos += 2
    srcv, dstv, rowb0, rowb1, zfv, gsem0, gsem1, acc = refs[pos:pos + 8]
    pos += 8
    if with_deg:
      onesv, z16v, dacc = refs[pos:pos + 3]; pos += 3
    rowb = (rowb0, rowb1)
    gsem = (gsem0, gsem1)

    c = lax.axis_index("c")
    s = lax.axis_index("s")
    base = s * RPS

    pltpu.sync_copy(zfeat_hbm, zfv)
    if with_deg:
      pltpu.sync_copy(z16_hbm, z16v)
      pltpu.sync_copy(ones_hbm, onesv)

    @pl.when(c == 0)
    def _():
      pltpu.sync_copy(su.at[s], srcv)
      pltpu.sync_copy(du.at[s], dstv)

    @pl.when(c == 1)
    def _():
      pltpu.sync_copy(si.at[s], srcv)
      pltpu.sync_copy(di.at[s], dstv)

    for p in range(num_passes):
      first = (p == 0)
      for k in range(NWB):
        pltpu.sync_copy(zfv, acc.at[pl.ds(base + k * CZ, CZ)])
        if with_deg and first:
          pltpu.sync_copy(z16v, dacc.at[pl.ds(base + k * CZ, CZ)])
      plsc.subcore_barrier()

      def run(x_hbm, do_deg):
        pltpu.async_copy(x_hbm.at[srcv.at[0]], rowb[0], gsem[0])

        def step(g, carry):
          for b in range(2):
            j = 2 * g + b
            nxt = (b + 1) % 2

            @pl.when(j + 1 < CH)
            def _():
              pltpu.async_copy(x_hbm.at[srcv.at[j + 1]], rowb[nxt], gsem[nxt])

            pltpu.make_async_copy(x_hbm.at[srcv.at[j]], rowb[b],
                                  gsem[b]).wait()
            pltpu.sync_copy(rowb[b], acc.at[dstv.at[j]], add=True)
            if do_deg:
              pltpu.sync_copy(onesv, dacc.at[dstv.at[j]], add=True)
          return carry

        lax.fori_loop(0, CH // 2, step, 0)

      @pl.when(c == 0)
      def _():
        run(xs_u[p], with_deg and first)

      @pl.when(c == 1)
      def _():
        run(xs_i[p], with_deg and first)

      plsc.subcore_barrier()

      def writeback(agg_hbm, deg_hbm):
        for k in range(NWB):
          off = base + k * CZ
          pltpu.sync_copy(acc.at[pl.ds(off, CZ)], zfv)
          pltpu.sync_copy(zfv, agg_hbm.at[pl.ds(off, CZ)])
          if deg_hbm is not None:
            pltpu.sync_copy(dacc.at[pl.ds(off, CZ)], z16v)
            pltpu.sync_copy(z16v, deg_hbm.at[pl.ds(off, CZ)])

      @pl.when(c == 0)
      def _():
        writeback(aggs_u[p], deg_u if (with_deg and first) else None)

      @pl.when(c == 1)
      def _():
        writeback(aggs_i[p], deg_i if (with_deg and first) else None)

      if p + 1 < num_passes:
        pltpu.sync_copy(zfeat_hbm, zfv)

  return sc_kernel


_sc_segsum_l1 = _make_sc_segsum(2, with_deg=True)
_sc_segsum_l2 = _make_sc_segsum(1, with_deg=False)


_BT = 1000
_GT = N // _BT


def _dot_t(a, w):
  return lax.dot_general(a, w, (((1,), (1,)), ((), ())),
                         preferred_element_type=_f32)


def _tc_mid_body(agg_u0, agg_u1, deg_u, xu, agg_i0, agg_i1, deg_i, xi,
                 uWl1, ubl1, uWr1, uWl2, ubl2, uWr2,
                 iWl1, ibl1, iWr1, iWl2, ibl2, iWr2,
                 p_u, r_u, p_i, r_i):
  def enc(agg0, agg1, deg, x, Wl1, bl1, Wr1, Wl2, bl2, Wr2, p_out, r_out):
    d = jnp.maximum(deg[...][:, :1], 1.0)
    mean = jnp.concatenate([agg0[...], agg1[...]], axis=1) / d
    h = jnp.maximum(_dot_t(mean, Wl1[...]) + bl1[...] + _dot_t(x[...], Wr1[...]),
                    0.0)
    p_out[...] = _dot_t(h, Wl2[...])
    r_out[...] = _dot_t(h, Wr2[...]) + bl2[...]

  enc(agg_u0, agg_u1, deg_u, xu, uWl1, ubl1, uWr1, uWl2, ubl2, uWr2, p_u, r_u)
  enc(agg_i0, agg_i1, deg_i, xi, iWl1, ibl1, iWr1, iWl2, ibl2, iWr2, p_i, r_i)


def _row_spec(d):
  return pl.BlockSpec((_BT, d), lambda i: (i, 0))


def _full_spec(shape):
  nd = len(shape)
  return pl.BlockSpec(shape, lambda i: (0,) * nd)


def _tc_mid(agg_u0, agg_u1, deg_u, xu, agg_i0, agg_i1, deg_i, xi, wu, wi):
  w_specs = [_full_spec(w.shape) for w in (wu + wi)]
  return pl.pallas_call(
      _tc_mid_body,
      grid=(_GT,),
      in_specs=[_row_spec(EMB), _row_spec(EMB), _row_spec(16), _row_spec(IN_DIM),
                _row_spec(EMB), _row_spec(EMB), _row_spec(16), _row_spec(IN_DIM)]
               + w_specs,
      out_specs=[_row_spec(EMB)] * 4,
      out_shape=[jax.ShapeDtypeStruct((N, EMB), _f32)] * 4,
  )(agg_u0, agg_u1, deg_u, xu, agg_i0, agg_i1, deg_i, xi, *wu, *wi)


def _tc_head_body(a2u, deg_u, ru, a2i, deg_i, ri, sW, sb, out):
  eu = a2u[...] / jnp.maximum(deg_u[...][:, :1], 1.0) + ru[...]
  ei = a2i[...] / jnp.maximum(deg_i[...][:, :1], 1.0) + ri[...]
  w = sW[...]
  z = _dot_t(eu, w[:, :EMB]) + _dot_t(ei, w[:, EMB:]) + sb[...]
  out[...] = 1.0 / (1.0 + jnp.exp(-z))


def _tc_head(a2u, deg_u, ru, a2i, deg_i, ri, sW, sb):
  return pl.pallas_call(
      _tc_head_body,
      grid=(_GT,),
      in_specs=[_row_spec(EMB), _row_spec(16), _row_spec(EMB),
                _row_spec(EMB), _row_spec(16), _row_spec(EMB),
                _full_spec((1, 2 * EMB)), _full_spec((1, 1))],
      out_specs=_row_spec(1),
      out_shape=jax.ShapeDtypeStruct((N, 1), _f32),
  )(a2u, deg_u, ru, a2i, deg_i, ri, sW, sb)


def kernel(user_x, item_x, user_edge_index, item_edge_index,
           u_Wl1, u_bl1, u_Wr1, u_Wl2, u_bl2, u_Wr2,
           i_Wl1, i_bl1, i_Wr1, i_Wl2, i_bl2, i_Wr2,
           s_W, s_b):
  def edges(ei):
    src = ei[0].astype(jnp.int32).reshape(NS, CH, C)
    dst = ei[1].astype(jnp.int32).reshape(NS, CH, C)
    return src, dst

  su, du = edges(user_edge_index)
  si, di = edges(item_edge_index)

  z64 = jnp.zeros((CZ, D), _f32)
  z16 = jnp.zeros((CZ, 16), _f32)
  ones16 = jnp.ones((C, 16), _f32)

  xu0, xu1 = user_x[:, :D], user_x[:, D:]
  xi0, xi1 = item_x[:, :D], item_x[:, D:]

  agg_u0, agg_u1, agg_i0, agg_i1, deg_u, deg_i = _sc_segsum_l1(
      xu0, xu1, xi0, xi1, su, du, si, di, z64, z16, ones16)

  wu = (u_Wl1, u_bl1.reshape(1, HID), u_Wr1,
        u_Wl2, u_bl2.reshape(1, EMB), u_Wr2)
  wi = (i_Wl1, i_bl1.reshape(1, HID), i_Wr1,
        i_Wl2, i_bl2.reshape(1, EMB), i_Wr2)
  p_u, r_u, p_i, r_i = _tc_mid(agg_u0, agg_u1, deg_u, user_x,
                               agg_i0, agg_i1, deg_i, item_x, wu, wi)

  agg2_u, agg2_i = _sc_segsum_l2(p_u, p_i, su, du, si, di, z64)

  return _tc_head(agg2_u, deg_u, r_u, agg2_i, deg_i, r_i,
                  s_W, s_b.reshape(1, 1))
